# baseline (speedup 1.0000x reference)
import functools

import jax
import jax.numpy as jnp
from jax import lax
from jax.experimental import pallas as pl
from jax.experimental.pallas import tpu as pltpu
from jax.experimental.pallas import tpu_sc as plsc

EMBED_DIM = 64
CHUNK = 128
NBUF = 5
NC = 2
NS = 16
NW = NC * NS


def _embed_body(cpw, seq_hbm, table_hbm, out_hbm, idx_v, bufs, *sems):
    c = lax.axis_index("c")
    s = lax.axis_index("s")
    wid = s * NC + c
    row0 = wid * cpw

    pltpu.sync_copy(seq_hbm.at[wid], idx_v)

    def outer(i, _):
        copies = []
        for b in range(NBUF):
            j = i * NBUF + b
            cp = pltpu.async_copy(
                table_hbm.at[idx_v.at[j]], bufs.at[b], sems[b]
            )
            copies.append((cp, j, b))
        for cp, j, b in copies:
            cp.wait()
            pltpu.sync_copy(
                bufs.at[b], out_hbm.at[pl.ds((row0 + j) * CHUNK, CHUNK)]
            )
        return 0

    lax.fori_loop(0, cpw // NBUF, outer, 0)


@functools.partial(jax.jit, static_argnums=(2,))
def _embed_call(seq3d, table, n_rows):
    cpw = n_rows // NW
    grid_kernel = pl.kernel(
        functools.partial(_embed_body, cpw),
        out_type=jax.ShapeDtypeStruct((n_rows * CHUNK, EMBED_DIM), jnp.float32),
        mesh=plsc.VectorSubcoreMesh(
            core_axis_name="c", subcore_axis_name="s",
            num_cores=NC, num_subcores=NS,
        ),
        scratch_types=[
            pltpu.VMEM((cpw, CHUNK), jnp.int32),
            pltpu.VMEM((NBUF, CHUNK, EMBED_DIM), jnp.float32),
        ] + [pltpu.SemaphoreType.DMA] * NBUF,
        compiler_params=pltpu.CompilerParams(use_tc_tiling_on_sc=False),
    )
    return grid_kernel(seq3d, table)


def kernel(seq, table):
    batch, seq_len = seq.shape
    total = batch * seq_len
    assert total % (NW * CHUNK * NBUF) == 0
    n_rows = total // CHUNK
    seq3d = seq.reshape(NW, n_rows // NW, CHUNK).astype(jnp.int32)
    out = _embed_call(seq3d, table, n_rows)
    return out.reshape(batch, seq_len, table.shape[1])


# speedup vs baseline: 1.0048x; 1.0048x over previous
import functools

import jax
import jax.numpy as jnp
from jax import lax
from jax.experimental import pallas as pl
from jax.experimental.pallas import tpu as pltpu
from jax.experimental.pallas import tpu_sc as plsc

EMBED_DIM = 64
CHUNK = 128
NBUF = 10
NC = 2
NS = 16
NW = NC * NS


def _embed_body(cpw, seq_hbm, table_hbm, out_hbm, idx_v, bufs, *sems):
    gsems = sems[:NBUF]
    osems = sems[NBUF:]
    c = lax.axis_index("c")
    s = lax.axis_index("s")
    wid = s * NC + c
    row0 = wid * cpw

    pltpu.sync_copy(seq_hbm.at[wid], idx_v)

    def gather(j, b):
        return pltpu.make_async_copy(
            table_hbm.at[idx_v.at[j]], bufs.at[b], gsems[b]
        )

    for b in range(NBUF):
        gather(b, b).start()

    def one_round(i, refill):
        writes = []
        for b in range(NBUF):
            j = i * NBUF + b
            gather(j, b).wait()
            writes.append(
                pltpu.async_copy(
                    bufs.at[b],
                    out_hbm.at[pl.ds((row0 + j) * CHUNK, CHUNK)],
                    osems[b],
                )
            )
        for b in range(NBUF):
            writes[b].wait()
            if refill:
                gather((i + 1) * NBUF + b, b).start()

    n_rounds = cpw // NBUF
    lax.fori_loop(0, n_rounds - 1, lambda i, _: (one_round(i, True), 0)[1], 0)
    one_round(n_rounds - 1, False)


@functools.partial(jax.jit, static_argnums=(2,))
def _embed_call(seq3d, table, n_rows):
    cpw = n_rows // NW
    grid_kernel = pl.kernel(
        functools.partial(_embed_body, cpw),
        out_type=jax.ShapeDtypeStruct((n_rows * CHUNK, EMBED_DIM), jnp.float32),
        mesh=plsc.VectorSubcoreMesh(
            core_axis_name="c", subcore_axis_name="s",
            num_cores=NC, num_subcores=NS,
        ),
        scratch_types=[
            pltpu.VMEM((cpw, CHUNK), jnp.int32),
            pltpu.VMEM((NBUF, CHUNK, EMBED_DIM), jnp.float32),
        ] + [pltpu.SemaphoreType.DMA] * (2 * NBUF),
        compiler_params=pltpu.CompilerParams(use_tc_tiling_on_sc=False),
    )
    return grid_kernel(seq3d, table)


def kernel(seq, table):
    batch, seq_len = seq.shape
    total = batch * seq_len
    assert total % (NW * CHUNK * NBUF) == 0
    n_rows = total // CHUNK
    seq3d = seq.reshape(NW, n_rows // NW, CHUNK).astype(jnp.int32)
    out = _embed_call(seq3d, table, n_rows)
    return out.reshape(batch, seq_len, table.shape[1])
